# Initial kernel scaffold; baseline (speedup 1.0000x reference)
import jax
import jax.numpy as jnp
from jax.experimental import pallas as pl


def kernel(x, grid0_params, grid1_params, W0, W1, W2):
    raise NotImplementedError("write your pallas kernel here")


# speedup vs baseline: 2.3610x; 2.3610x over previous
import functools

import jax
import jax.numpy as jnp
from jax import lax
from jax.experimental import pallas as pl
from jax.experimental.pallas import tpu as pltpu
from jax.experimental.pallas import tpu_sc as plsc

_N_FREQ = 12
_NUM_LODS = 8
_FPL = 2
_BATCH = 524288
_GRID_BASES = (16, 8)
_NLV = 8

_NC = 2
_NS = 16
_L = 16
_NW = _NC * _NS
_PTS_W = _BATCH // _NW
_CH = 128
_STEPS = _PTS_W // _CH


def _sc_feats_body(u_hbm, v_hbm, l_hbm, t0_hbm, t1_hbm,
                   o0_hbm, o1_hbm, o2_hbm, o3_hbm,
                   u_v, v_v, l_v, idx_v, val_v, w_v, out_v, sem):
    wid = lax.axis_index("s") * _NC + lax.axis_index("c")
    wbase = wid * _PTS_W

    def step(s, carry):
        base = wbase + s * _CH
        pltpu.sync_copy(u_hbm.at[pl.ds(base, _CH)], u_v)
        pltpu.sync_copy(v_hbm.at[pl.ds(base, _CH)], v_v)
        pltpu.sync_copy(l_hbm.at[pl.ds(base, _CH)], l_v)

        for g in range(_CH // _L):
            sl = pl.ds(g * _L, _L)
            u = u_v[sl]
            v = v_v[sl]
            lod = l_v[sl]
            mips = lod * float(_NUM_LODS - 1)
            t = (float(_NLV - 1) - jnp.minimum(mips, float(_NLV - 1))) * 2.0
            for gi in range(2):
                r0 = _GRID_BASES[gi]
                for k in range(2):
                    col = (t + float(k)).astype(jnp.int32)
                    lvl = lax.shift_right_logical(col, 1)
                    feat = lax.bitwise_and(col, 1)
                    res = lax.shift_left(jnp.full((_L,), r0, jnp.int32), lvl)
                    scale = res.astype(jnp.float32) - 1.0
                    pow4m1 = lax.shift_left(jnp.full((_L,), 1, jnp.int32),
                                            lvl * 2) - 1
                    offs = (r0 * r0) * lax.bitwise_and(pow4m1, 0x55555555)
                    px = u * scale + 0.5
                    py = v * scale + 0.5
                    p0xi = px.astype(jnp.int32)
                    p0yi = py.astype(jnp.int32)
                    wx = px - p0xi.astype(jnp.float32)
                    wy = py - p0yi.astype(jnp.float32)
                    rm1 = res - 1
                    p0x = jnp.clip(p0xi, 0, rm1)
                    p0y = jnp.clip(p0yi, 0, rm1)
                    p1x = jnp.minimum(p0x + 1, rm1)
                    p1y = jnp.minimum(p0y + 1, rm1)
                    rowa = offs + p0y * res
                    rowb = offs + p1y * res
                    r = gi * 8 + k * 4
                    idx_v[r + 0, sl] = 2 * (rowa + p0x) + feat
                    idx_v[r + 1, sl] = 2 * (rowa + p1x) + feat
                    idx_v[r + 2, sl] = 2 * (rowb + p0x) + feat
                    idx_v[r + 3, sl] = 2 * (rowb + p1x) + feat
                    w = gi * 4 + k * 2
                    w_v[w + 0, sl] = wx
                    w_v[w + 1, sl] = wy

        copies = []
        for r in range(8):
            copies.append(pltpu.async_copy(t0_hbm.at[idx_v.at[r]],
                                           val_v.at[r], sem))
        for r in range(8, 16):
            copies.append(pltpu.async_copy(t1_hbm.at[idx_v.at[r]],
                                           val_v.at[r], sem))
        for c in copies:
            c.wait()

        for g in range(_CH // _L):
            sl = pl.ds(g * _L, _L)
            for gi in range(2):
                for k in range(2):
                    r = gi * 8 + k * 4
                    f00 = val_v[r + 0, sl]
                    f10 = val_v[r + 1, sl]
                    f01 = val_v[r + 2, sl]
                    f11 = val_v[r + 3, sl]
                    w = gi * 4 + k * 2
                    wx = w_v[w + 0, sl]
                    wy = w_v[w + 1, sl]
                    omx = 1.0 - wx
                    omy = 1.0 - wy
                    out_v[gi * 2 + k, sl] = (f00 * omx * omy + f10 * wx * omy
                                             + f01 * omx * wy + f11 * wx * wy)

        pltpu.sync_copy(out_v.at[0], o0_hbm.at[pl.ds(base, _CH)])
        pltpu.sync_copy(out_v.at[1], o1_hbm.at[pl.ds(base, _CH)])
        pltpu.sync_copy(out_v.at[2], o2_hbm.at[pl.ds(base, _CH)])
        pltpu.sync_copy(out_v.at[3], o3_hbm.at[pl.ds(base, _CH)])
        return carry

    lax.fori_loop(0, _STEPS, step, 0)


def _sc_feats(u, v, lod, t0, t1):
    out = jax.ShapeDtypeStruct((_BATCH,), jnp.float32)
    mesh = plsc.VectorSubcoreMesh(core_axis_name="c", subcore_axis_name="s")
    return pl.kernel(
        _sc_feats_body,
        out_type=[out, out, out, out],
        mesh=mesh,
        scratch_types=[
            pltpu.VMEM((_CH,), jnp.float32),
            pltpu.VMEM((_CH,), jnp.float32),
            pltpu.VMEM((_CH,), jnp.float32),
            pltpu.VMEM((16, _CH), jnp.int32),
            pltpu.VMEM((16, _CH), jnp.float32),
            pltpu.VMEM((8, _CH), jnp.float32),
            pltpu.VMEM((4, _CH), jnp.float32),
            pltpu.SemaphoreType.DMA,
        ],
    )(u, v, lod, t0, t1)


_BM = 2048


def _mlp_body(x_ref, f_ref, w0_ref, w1_ref, w2_ref, o_ref):
    x = x_ref[...]
    u = x[:, 0:1]
    v = x[:, 1:2]
    lod = x[:, 2:3]
    kf = lax.broadcasted_iota(jnp.int32, (_BM, _N_FREQ), 1).astype(jnp.float32)
    freqs = jnp.exp2(kf)

    def tri(w):
        return jnp.abs(w - jnp.floor(w + 0.5)) * 4.0 - 1.0

    pe = jnp.concatenate([tri(u * freqs), tri(v * freqs)], axis=1)
    h = jnp.concatenate(
        [pe, f_ref[...], lod, jnp.zeros((_BM, 3), jnp.float32)], axis=1)

    def lrelu(a):
        return jnp.where(a >= 0, a, 0.01 * a)

    h = lrelu(jnp.dot(h, w0_ref[...], preferred_element_type=jnp.float32))
    h = lrelu(jnp.dot(h, w1_ref[...], preferred_element_type=jnp.float32))
    h = lrelu(jnp.dot(h, w2_ref[...], preferred_element_type=jnp.float32))
    o_ref[...] = h


def _mlp(x, feats, w0p, w1, w2p):
    grid = (_BATCH // _BM,)
    return pl.pallas_call(
        _mlp_body,
        grid=grid,
        in_specs=[
            pl.BlockSpec((_BM, 3), lambda i: (i, 0)),
            pl.BlockSpec((_BM, 4), lambda i: (i, 0)),
            pl.BlockSpec((32, 64), lambda i: (0, 0)),
            pl.BlockSpec((64, 64), lambda i: (0, 0)),
            pl.BlockSpec((64, 8), lambda i: (0, 0)),
        ],
        out_specs=pl.BlockSpec((_BM, 8), lambda i: (i, 0)),
        out_shape=jax.ShapeDtypeStruct((_BATCH, 8), jnp.float32),
    )(x, feats, w0p, w1, w2p)


@jax.jit
def kernel(x, grid0_params, grid1_params, W0, W1, W2):
    u = x[:, 0]
    v = x[:, 1]
    lod = x[:, 2]
    t0 = grid0_params.reshape(-1)
    t1 = grid1_params.reshape(-1)
    f0a, f0b, f1a, f1b = _sc_feats(u, v, lod, t0, t1)
    feats = jnp.stack([f0a, f0b, f1a, f1b], axis=1)
    w0p = jnp.pad(W0, ((0, 3), (0, 0)))
    w2p = jnp.pad(W2, ((0, 0), (0, 5)))
    out = _mlp(x, feats, w0p, W1, w2p)
    return out[:, :3]


# speedup vs baseline: 5.6395x; 2.3886x over previous
import functools

import jax
import jax.numpy as jnp
from jax import lax
from jax.experimental import pallas as pl
from jax.experimental.pallas import tpu as pltpu
from jax.experimental.pallas import tpu_sc as plsc

_N_FREQ = 12
_NUM_LODS = 8
_FPL = 2
_BATCH = 524288
_GRID_BASES = (16, 8)
_NLV = 8

_NC = 2
_NS = 16
_L = 16
_NW = _NC * _NS
_PTS_W = _BATCH // _NW
_CH = 128
_STEPS = _PTS_W // _CH


def _sc_feats_body(u_hbm, v_hbm, l_hbm, t0_hbm, t1_hbm,
                   o0_hbm, o1_hbm, o2_hbm, o3_hbm,
                   u_v, v_v, l_v, idx_v, val_v, w_v, out_v, sem):
    wid = lax.axis_index("s") * _NC + lax.axis_index("c")
    wbase = wid * _PTS_W

    def step(s, carry):
        base = wbase + s * _CH
        pltpu.sync_copy(u_hbm.at[pl.ds(base, _CH)], u_v)
        pltpu.sync_copy(v_hbm.at[pl.ds(base, _CH)], v_v)
        pltpu.sync_copy(l_hbm.at[pl.ds(base, _CH)], l_v)

        for g in range(_CH // _L):
            sl = pl.ds(g * _L, _L)
            u = u_v[sl]
            v = v_v[sl]
            lod = l_v[sl]
            mips = lod * float(_NUM_LODS - 1)
            t = (float(_NLV - 1) - jnp.minimum(mips, float(_NLV - 1))) * 2.0
            for gi in range(2):
                r0 = _GRID_BASES[gi]
                for k in range(2):
                    col = (t + float(k)).astype(jnp.int32)
                    lvl = lax.shift_right_logical(col, 1)
                    feat = lax.bitwise_and(col, 1)
                    res = lax.shift_left(jnp.full((_L,), r0, jnp.int32), lvl)
                    scale = res.astype(jnp.float32) - 1.0
                    pow4m1 = lax.shift_left(jnp.full((_L,), 1, jnp.int32),
                                            lvl * 2) - 1
                    offs = (r0 * r0) * lax.bitwise_and(pow4m1, 0x55555555)
                    px = u * scale + 0.5
                    py = v * scale + 0.5
                    p0xi = px.astype(jnp.int32)
                    p0yi = py.astype(jnp.int32)
                    wx = px - p0xi.astype(jnp.float32)
                    wy = py - p0yi.astype(jnp.float32)
                    rm1 = res - 1
                    p0x = jnp.clip(p0xi, 0, rm1)
                    p0y = jnp.clip(p0yi, 0, rm1)
                    p1x = jnp.minimum(p0x + 1, rm1)
                    p1y = jnp.minimum(p0y + 1, rm1)
                    rowa = offs + p0y * res
                    rowb = offs + p1y * res
                    rows = (rowa + p0x, rowa + p1x, rowb + p0x, rowb + p1x)
                    r = gi * 8 + k * 4
                    for j, row in enumerate(rows):
                        if gi == 0:
                            e = (lax.shift_left(
                                    lax.shift_right_logical(row, 7), 8)
                                 + lax.shift_left(feat, 7)
                                 + lax.bitwise_and(row, 127))
                        else:
                            e = lax.shift_left(row, 1) + feat
                        idx_v[r + j, sl] = e
                    w = gi * 4 + k * 2
                    w_v[w + 0, sl] = wx
                    w_v[w + 1, sl] = wy

        copies = []
        for r in range(8):
            copies.append(pltpu.async_copy(t0_hbm.at[idx_v.at[r]],
                                           val_v.at[r], sem))
        for r in range(8, 16):
            copies.append(pltpu.async_copy(t1_hbm.at[idx_v.at[r]],
                                           val_v.at[r], sem))
        for c in copies:
            c.wait()

        for g in range(_CH // _L):
            sl = pl.ds(g * _L, _L)
            for gi in range(2):
                for k in range(2):
                    r = gi * 8 + k * 4
                    f00 = val_v[r + 0, sl]
                    f10 = val_v[r + 1, sl]
                    f01 = val_v[r + 2, sl]
                    f11 = val_v[r + 3, sl]
                    w = gi * 4 + k * 2
                    wx = w_v[w + 0, sl]
                    wy = w_v[w + 1, sl]
                    omx = 1.0 - wx
                    omy = 1.0 - wy
                    out_v[gi * 2 + k, sl] = (f00 * omx * omy + f10 * wx * omy
                                             + f01 * omx * wy + f11 * wx * wy)

        pltpu.sync_copy(out_v.at[0], o0_hbm.at[pl.ds(base, _CH)])
        pltpu.sync_copy(out_v.at[1], o1_hbm.at[pl.ds(base, _CH)])
        pltpu.sync_copy(out_v.at[2], o2_hbm.at[pl.ds(base, _CH)])
        pltpu.sync_copy(out_v.at[3], o3_hbm.at[pl.ds(base, _CH)])
        return carry

    lax.fori_loop(0, _STEPS, step, 0)


def _sc_feats(u, v, lod, t0, t1):
    out = jax.ShapeDtypeStruct((_BATCH,), jnp.float32)
    mesh = plsc.VectorSubcoreMesh(core_axis_name="c", subcore_axis_name="s")
    return pl.kernel(
        _sc_feats_body,
        out_type=[out, out, out, out],
        mesh=mesh,
        scratch_types=[
            pltpu.VMEM((_CH,), jnp.float32),
            pltpu.VMEM((_CH,), jnp.float32),
            pltpu.VMEM((_CH,), jnp.float32),
            pltpu.VMEM((16, _CH), jnp.int32),
            pltpu.VMEM((16, _CH), jnp.float32),
            pltpu.VMEM((8, _CH), jnp.float32),
            pltpu.VMEM((4, _CH), jnp.float32),
            pltpu.SemaphoreType.DMA,
        ],
    )(u, v, lod, t0, t1)


_BM = 2048


def _mlp_body(x_ref, f_ref, w0_ref, w1_ref, w2_ref, o_ref):
    x = x_ref[...]
    u = x[:, 0:1]
    v = x[:, 1:2]
    lod = x[:, 2:3]
    kf = lax.broadcasted_iota(jnp.int32, (_BM, _N_FREQ), 1).astype(jnp.float32)
    freqs = jnp.exp2(kf)

    def tri(w):
        return jnp.abs(w - jnp.floor(w + 0.5)) * 4.0 - 1.0

    pe = jnp.concatenate([tri(u * freqs), tri(v * freqs)], axis=1)
    h = jnp.concatenate(
        [pe, f_ref[...], lod, jnp.zeros((_BM, 3), jnp.float32)], axis=1)

    def lrelu(a):
        return jnp.where(a >= 0, a, 0.01 * a)

    h = lrelu(jnp.dot(h, w0_ref[...], preferred_element_type=jnp.float32))
    h = lrelu(jnp.dot(h, w1_ref[...], preferred_element_type=jnp.float32))
    h = lrelu(jnp.dot(h, w2_ref[...], preferred_element_type=jnp.float32))
    o_ref[...] = h


def _mlp(x, feats, w0p, w1, w2p):
    grid = (_BATCH // _BM,)
    return pl.pallas_call(
        _mlp_body,
        grid=grid,
        in_specs=[
            pl.BlockSpec((_BM, 3), lambda i: (i, 0)),
            pl.BlockSpec((_BM, 4), lambda i: (i, 0)),
            pl.BlockSpec((32, 64), lambda i: (0, 0)),
            pl.BlockSpec((64, 64), lambda i: (0, 0)),
            pl.BlockSpec((64, 8), lambda i: (0, 0)),
        ],
        out_specs=pl.BlockSpec((_BM, 8), lambda i: (i, 0)),
        out_shape=jax.ShapeDtypeStruct((_BATCH, 8), jnp.float32),
    )(x, feats, w0p, w1, w2p)


@jax.jit
def kernel(x, grid0_params, grid1_params, W0, W1, W2):
    u = x[:, 0]
    v = x[:, 1]
    lod = x[:, 2]
    n0 = grid0_params.shape[0]
    t0 = grid0_params.reshape(n0 // 128, 128, 2).transpose(0, 2, 1).reshape(-1)
    t1 = grid1_params.reshape(-1)
    f0a, f0b, f1a, f1b = _sc_feats(u, v, lod, t0, t1)
    feats = jnp.stack([f0a, f0b, f1a, f1b], axis=1)
    w0p = jnp.pad(W0, ((0, 3), (0, 0)))
    w2p = jnp.pad(W2, ((0, 0), (0, 5)))
    out = _mlp(x, feats, w0p, W1, w2p)
    return out[:, :3]


# speedup vs baseline: 9.8231x; 1.7418x over previous
import functools

import jax
import jax.numpy as jnp
from jax import lax
from jax.experimental import pallas as pl
from jax.experimental.pallas import tpu as pltpu
from jax.experimental.pallas import tpu_sc as plsc

_N_FREQ = 12
_NUM_LODS = 8
_FPL = 2
_BATCH = 524288
_GRID_BASES = (16, 8)
_NLV = 8

_NC = 2
_NS = 16
_L = 16
_NW = _NC * _NS
_PTS_W = _BATCH // _NW
_CH = 128
_STEPS = _PTS_W // _CH


def _sc_feats_body(u_hbm, v_hbm, l_hbm, t0_hbm, t1_hbm,
                   o0_hbm, o1_hbm, o2_hbm, o3_hbm,
                   u_v, v_v, l_v, idx_v, val_v, w_v, out_v, sem):
    wid = lax.axis_index("s") * _NC + lax.axis_index("c")
    wbase = wid * _PTS_W

    def step(s, carry):
        base = wbase + s * _CH
        pltpu.sync_copy(u_hbm.at[pl.ds(base, _CH)], u_v)
        pltpu.sync_copy(v_hbm.at[pl.ds(base, _CH)], v_v)
        pltpu.sync_copy(l_hbm.at[pl.ds(base, _CH)], l_v)

        for g in range(_CH // _L):
            sl = pl.ds(g * _L, _L)
            u = u_v[sl]
            v = v_v[sl]
            lod = l_v[sl]
            mips = lod * float(_NUM_LODS - 1)
            t = (float(_NLV - 1) - jnp.minimum(mips, float(_NLV - 1))) * 2.0
            for gi in range(2):
                r0 = _GRID_BASES[gi]
                for k in range(2):
                    col = (t + float(k)).astype(jnp.int32)
                    lvl = lax.shift_right_logical(col, 1)
                    feat = lax.bitwise_and(col, 1)
                    res = lax.shift_left(jnp.full((_L,), r0, jnp.int32), lvl)
                    scale = res.astype(jnp.float32) - 1.0
                    pow4m1 = lax.shift_left(jnp.full((_L,), 1, jnp.int32),
                                            lvl * 2) - 1
                    offs = (r0 * r0) * lax.bitwise_and(pow4m1, 0x55555555)
                    px = u * scale + 0.5
                    py = v * scale + 0.5
                    p0xi = px.astype(jnp.int32)
                    p0yi = py.astype(jnp.int32)
                    wx = px - p0xi.astype(jnp.float32)
                    wy = py - p0yi.astype(jnp.float32)
                    rm1 = res - 1
                    p0x = jnp.clip(p0xi, 0, rm1)
                    p0y = jnp.clip(p0yi, 0, rm1)
                    p1x = jnp.minimum(p0x + 1, rm1)
                    p1y = jnp.minimum(p0y + 1, rm1)
                    rowa = offs + p0y * res
                    rowb = offs + p1y * res
                    rows = (rowa + p0x, rowa + p1x, rowb + p0x, rowb + p1x)
                    r = gi * 8 + k * 4
                    for j, row in enumerate(rows):
                        e = (lax.shift_left(
                                lax.shift_right_logical(row, 7), 8)
                             + lax.shift_left(feat, 7)
                             + lax.bitwise_and(row, 127))
                        idx_v[r + j, sl] = e
                    w = gi * 4 + k * 2
                    w_v[w + 0, sl] = wx
                    w_v[w + 1, sl] = wy

        copies = []
        for r in range(8):
            copies.append(pltpu.async_copy(t0_hbm.at[idx_v.at[r]],
                                           val_v.at[r], sem))
        for r in range(8, 16):
            copies.append(pltpu.async_copy(t1_hbm.at[idx_v.at[r]],
                                           val_v.at[r], sem))
        for c in copies:
            c.wait()

        for g in range(_CH // _L):
            sl = pl.ds(g * _L, _L)
            for gi in range(2):
                for k in range(2):
                    r = gi * 8 + k * 4
                    f00 = val_v[r + 0, sl]
                    f10 = val_v[r + 1, sl]
                    f01 = val_v[r + 2, sl]
                    f11 = val_v[r + 3, sl]
                    w = gi * 4 + k * 2
                    wx = w_v[w + 0, sl]
                    wy = w_v[w + 1, sl]
                    omx = 1.0 - wx
                    omy = 1.0 - wy
                    out_v[gi * 2 + k, sl] = (f00 * omx * omy + f10 * wx * omy
                                             + f01 * omx * wy + f11 * wx * wy)

        pltpu.sync_copy(out_v.at[0], o0_hbm.at[pl.ds(base, _CH)])
        pltpu.sync_copy(out_v.at[1], o1_hbm.at[pl.ds(base, _CH)])
        pltpu.sync_copy(out_v.at[2], o2_hbm.at[pl.ds(base, _CH)])
        pltpu.sync_copy(out_v.at[3], o3_hbm.at[pl.ds(base, _CH)])
        return carry

    lax.fori_loop(0, _STEPS, step, 0)


def _sc_feats(u, v, lod, t0, t1):
    out = jax.ShapeDtypeStruct((_BATCH,), jnp.float32)
    mesh = plsc.VectorSubcoreMesh(core_axis_name="c", subcore_axis_name="s")
    return pl.kernel(
        _sc_feats_body,
        out_type=[out, out, out, out],
        mesh=mesh,
        scratch_types=[
            pltpu.VMEM((_CH,), jnp.float32),
            pltpu.VMEM((_CH,), jnp.float32),
            pltpu.VMEM((_CH,), jnp.float32),
            pltpu.VMEM((16, _CH), jnp.int32),
            pltpu.VMEM((16, _CH), jnp.float32),
            pltpu.VMEM((8, _CH), jnp.float32),
            pltpu.VMEM((4, _CH), jnp.float32),
            pltpu.SemaphoreType.DMA,
        ],
    )(u, v, lod, t0, t1)


_BM = 2048


def _mlp_body(x_ref, f_ref, w0_ref, w1_ref, w2_ref, o_ref):
    x = x_ref[...]
    u = x[:, 0:1]
    v = x[:, 1:2]
    lod = x[:, 2:3]
    kf = lax.broadcasted_iota(jnp.int32, (_BM, _N_FREQ), 1).astype(jnp.float32)
    freqs = jnp.exp2(kf)

    def tri(w):
        return jnp.abs(w - jnp.floor(w + 0.5)) * 4.0 - 1.0

    pe = jnp.concatenate([tri(u * freqs), tri(v * freqs)], axis=1)
    h = jnp.concatenate(
        [pe, f_ref[...], lod, jnp.zeros((_BM, 3), jnp.float32)], axis=1)

    def lrelu(a):
        return jnp.where(a >= 0, a, 0.01 * a)

    h = lrelu(jnp.dot(h, w0_ref[...], preferred_element_type=jnp.float32))
    h = lrelu(jnp.dot(h, w1_ref[...], preferred_element_type=jnp.float32))
    h = lrelu(jnp.dot(h, w2_ref[...], preferred_element_type=jnp.float32))
    o_ref[...] = h


def _mlp(x, feats, w0p, w1, w2p):
    grid = (_BATCH // _BM,)
    return pl.pallas_call(
        _mlp_body,
        grid=grid,
        in_specs=[
            pl.BlockSpec((_BM, 3), lambda i: (i, 0)),
            pl.BlockSpec((_BM, 4), lambda i: (i, 0)),
            pl.BlockSpec((32, 64), lambda i: (0, 0)),
            pl.BlockSpec((64, 64), lambda i: (0, 0)),
            pl.BlockSpec((64, 8), lambda i: (0, 0)),
        ],
        out_specs=pl.BlockSpec((_BM, 8), lambda i: (i, 0)),
        out_shape=jax.ShapeDtypeStruct((_BATCH, 8), jnp.float32),
    )(x, feats, w0p, w1, w2p)


@jax.jit
def kernel(x, grid0_params, grid1_params, W0, W1, W2):
    u = x[:, 0]
    v = x[:, 1]
    lod = x[:, 2]
    def tiled_flat(t):
        n = t.shape[0]
        if n % 128:
            t = jnp.pad(t, ((0, 128 - n % 128), (0, 0)))
            n = t.shape[0]
        return t.reshape(n // 128, 128, 2).transpose(0, 2, 1).reshape(-1)

    t0 = tiled_flat(grid0_params)
    t1 = tiled_flat(grid1_params)
    f0a, f0b, f1a, f1b = _sc_feats(u, v, lod, t0, t1)
    feats = jnp.stack([f0a, f0b, f1a, f1b], axis=1)
    w0p = jnp.pad(W0, ((0, 3), (0, 0)))
    w2p = jnp.pad(W2, ((0, 0), (0, 5)))
    out = _mlp(x, feats, w0p, W1, w2p)
    return out[:, :3]


# speedup vs baseline: 9.8264x; 1.0003x over previous
import functools

import jax
import jax.numpy as jnp
from jax import lax
from jax.experimental import pallas as pl
from jax.experimental.pallas import tpu as pltpu
from jax.experimental.pallas import tpu_sc as plsc

_N_FREQ = 12
_NUM_LODS = 8
_FPL = 2
_BATCH = 524288
_GRID_BASES = (16, 8)
_NLV = 8

_NC = 2
_NS = 16
_L = 16
_NW = _NC * _NS
_PTS_W = _BATCH // _NW
_CH = 128
_BLK = 1024
_NCH = _BLK // _CH
_NBLK = _PTS_W // _BLK


def _sc_feats_body(u_hbm, v_hbm, l_hbm, t0_hbm, t1_hbm,
                   o0_hbm, o1_hbm, o2_hbm, o3_hbm,
                   u_v, v_v, l_v, idx_v, val_v, w_v, out_v,
                   sem_g, sem_x, sem_o):
    wid = lax.axis_index("s") * _NC + lax.axis_index("c")
    wbase = wid * _PTS_W
    xin = ((u_hbm, u_v), (v_hbm, v_v), (l_hbm, l_v))
    outs = (o0_hbm, o1_hbm, o2_hbm, o3_hbm)

    def start_x(b, p):
        base = wbase + b * _BLK
        for hb, a in xin:
            pltpu.async_copy(hb.at[pl.ds(base, _BLK)], a.at[p], sem_x)

    def wait_x():
        for hb, a in xin:
            pltpu.make_async_copy(hb.at[pl.ds(0, _BLK)], a.at[0],
                                  sem_x).wait()

    start_x(0, 0)

    def blk(b, carry):
        p = lax.bitwise_and(b, 1)
        base = wbase + b * _BLK
        wait_x()
        start_x(jnp.minimum(b + 1, _NBLK - 1), 1 - p)

        def ph1(c, _):
            for g in range(_CH // _L):
                sl = pl.ds(c * _CH + g * _L, _L)
                slc = pl.ds(g * _L, _L)
                u = u_v[p, sl]
                v = v_v[p, sl]
                lod = l_v[p, sl]
                mips = lod * float(_NUM_LODS - 1)
                t = (float(_NLV - 1)
                     - jnp.minimum(mips, float(_NLV - 1))) * 2.0
                for gi in range(2):
                    r0 = _GRID_BASES[gi]
                    for k in range(2):
                        col = (t + float(k)).astype(jnp.int32)
                        lvl = lax.shift_right_logical(col, 1)
                        feat = lax.bitwise_and(col, 1)
                        res = lax.shift_left(
                            jnp.full((_L,), r0, jnp.int32), lvl)
                        scale = res.astype(jnp.float32) - 1.0
                        pow4m1 = lax.shift_left(
                            jnp.full((_L,), 1, jnp.int32), lvl * 2) - 1
                        offs = (r0 * r0) * lax.bitwise_and(pow4m1,
                                                           0x55555555)
                        px = u * scale + 0.5
                        py = v * scale + 0.5
                        p0xi = px.astype(jnp.int32)
                        p0yi = py.astype(jnp.int32)
                        wx = px - p0xi.astype(jnp.float32)
                        wy = py - p0yi.astype(jnp.float32)
                        rm1 = res - 1
                        p0x = jnp.clip(p0xi, 0, rm1)
                        p0y = jnp.clip(p0yi, 0, rm1)
                        p1x = jnp.minimum(p0x + 1, rm1)
                        p1y = jnp.minimum(p0y + 1, rm1)
                        rowa = offs + p0y * res
                        rowb = offs + p1y * res
                        rows = (rowa + p0x, rowa + p1x,
                                rowb + p0x, rowb + p1x)
                        r = gi * 8 + k * 4
                        for j, row in enumerate(rows):
                            e = (lax.shift_left(
                                    lax.shift_right_logical(row, 7), 8)
                                 + lax.shift_left(feat, 7)
                                 + lax.bitwise_and(row, 127))
                            idx_v[c, r + j, slc] = e
                        w = gi * 4 + k * 2
                        w_v[c, w + 0, slc] = wx
                        w_v[c, w + 1, slc] = wy
            for r in range(8):
                pltpu.async_copy(t0_hbm.at[idx_v.at[c].at[r]],
                                 val_v.at[c].at[r], sem_g)
            for r in range(8, 16):
                pltpu.async_copy(t1_hbm.at[idx_v.at[c].at[r]],
                                 val_v.at[c].at[r], sem_g)
            return _

        lax.fori_loop(0, _NCH, ph1, 0)

        def drain(c, _):
            for r in range(16):
                pltpu.make_async_copy(t0_hbm.at[idx_v.at[0].at[0]],
                                      val_v.at[0].at[0], sem_g).wait()
            return _

        lax.fori_loop(0, _NCH, drain, 0)

        @pl.when(b >= 2)
        def _wait_out():
            for j in range(4):
                pltpu.make_async_copy(out_v.at[0].at[0],
                                      o0_hbm.at[pl.ds(0, _BLK)],
                                      sem_o).wait()

        def ph2(c, _):
            for g in range(_CH // _L):
                sl = pl.ds(c * _CH + g * _L, _L)
                slc = pl.ds(g * _L, _L)
                for gi in range(2):
                    for k in range(2):
                        r = gi * 8 + k * 4
                        f00 = val_v[c, r + 0, slc]
                        f10 = val_v[c, r + 1, slc]
                        f01 = val_v[c, r + 2, slc]
                        f11 = val_v[c, r + 3, slc]
                        w = gi * 4 + k * 2
                        wx = w_v[c, w + 0, slc]
                        wy = w_v[c, w + 1, slc]
                        omx = 1.0 - wx
                        omy = 1.0 - wy
                        out_v[p, gi * 2 + k, sl] = (
                            f00 * omx * omy + f10 * wx * omy
                            + f01 * omx * wy + f11 * wx * wy)
            return _

        lax.fori_loop(0, _NCH, ph2, 0)

        for j, o_hbm in enumerate(outs):
            pltpu.async_copy(out_v.at[p].at[j],
                             o_hbm.at[pl.ds(base, _BLK)], sem_o)
        return carry

    lax.fori_loop(0, _NBLK, blk, 0)

    for _ in range(8):
        pltpu.make_async_copy(out_v.at[0].at[0], o0_hbm.at[pl.ds(0, _BLK)],
                              sem_o).wait()
    wait_x()


def _sc_feats(u, v, lod, t0, t1):
    out = jax.ShapeDtypeStruct((_BATCH,), jnp.float32)
    mesh = plsc.VectorSubcoreMesh(core_axis_name="c", subcore_axis_name="s")
    return pl.kernel(
        _sc_feats_body,
        out_type=[out, out, out, out],
        mesh=mesh,
        scratch_types=[
            pltpu.VMEM((2, _BLK), jnp.float32),
            pltpu.VMEM((2, _BLK), jnp.float32),
            pltpu.VMEM((2, _BLK), jnp.float32),
            pltpu.VMEM((_NCH, 16, _CH), jnp.int32),
            pltpu.VMEM((_NCH, 16, _CH), jnp.float32),
            pltpu.VMEM((_NCH, 8, _CH), jnp.float32),
            pltpu.VMEM((2, 4, _BLK), jnp.float32),
            pltpu.SemaphoreType.DMA,
            pltpu.SemaphoreType.DMA,
            pltpu.SemaphoreType.DMA,
        ],
    )(u, v, lod, t0, t1)


_BM = 2048


def _mlp_body(x_ref, f_ref, w0_ref, w1_ref, w2_ref, o_ref):
    x = x_ref[...]
    u = x[:, 0:1]
    v = x[:, 1:2]
    lod = x[:, 2:3]
    kf = lax.broadcasted_iota(jnp.int32, (_BM, _N_FREQ), 1).astype(jnp.float32)
    freqs = jnp.exp2(kf)

    def tri(w):
        return jnp.abs(w - jnp.floor(w + 0.5)) * 4.0 - 1.0

    pe = jnp.concatenate([tri(u * freqs), tri(v * freqs)], axis=1)
    h = jnp.concatenate(
        [pe, f_ref[...], lod, jnp.zeros((_BM, 3), jnp.float32)], axis=1)

    def lrelu(a):
        return jnp.where(a >= 0, a, 0.01 * a)

    h = lrelu(jnp.dot(h, w0_ref[...], preferred_element_type=jnp.float32))
    h = lrelu(jnp.dot(h, w1_ref[...], preferred_element_type=jnp.float32))
    h = lrelu(jnp.dot(h, w2_ref[...], preferred_element_type=jnp.float32))
    o_ref[...] = h


def _mlp(x, feats, w0p, w1, w2p):
    grid = (_BATCH // _BM,)
    return pl.pallas_call(
        _mlp_body,
        grid=grid,
        in_specs=[
            pl.BlockSpec((_BM, 3), lambda i: (i, 0)),
            pl.BlockSpec((_BM, 4), lambda i: (i, 0)),
            pl.BlockSpec((32, 64), lambda i: (0, 0)),
            pl.BlockSpec((64, 64), lambda i: (0, 0)),
            pl.BlockSpec((64, 8), lambda i: (0, 0)),
        ],
        out_specs=pl.BlockSpec((_BM, 8), lambda i: (i, 0)),
        out_shape=jax.ShapeDtypeStruct((_BATCH, 8), jnp.float32),
    )(x, feats, w0p, w1, w2p)


@jax.jit
def kernel(x, grid0_params, grid1_params, W0, W1, W2):
    u = x[:, 0]
    v = x[:, 1]
    lod = x[:, 2]
    def tiled_flat(t):
        n = t.shape[0]
        if n % 128:
            t = jnp.pad(t, ((0, 128 - n % 128), (0, 0)))
            n = t.shape[0]
        return t.reshape(n // 128, 128, 2).transpose(0, 2, 1).reshape(-1)

    t0 = tiled_flat(grid0_params)
    t1 = tiled_flat(grid1_params)
    f0a, f0b, f1a, f1b = _sc_feats(u, v, lod, t0, t1)
    feats = jnp.stack([f0a, f0b, f1a, f1b], axis=1)
    w0p = jnp.pad(W0, ((0, 3), (0, 0)))
    w2p = jnp.pad(W2, ((0, 0), (0, 5)))
    out = _mlp(x, feats, w0p, W1, w2p)
    return out[:, :3]


# speedup vs baseline: 10.1253x; 1.0304x over previous
import functools

import jax
import jax.numpy as jnp
from jax import lax
from jax.experimental import pallas as pl
from jax.experimental.pallas import tpu as pltpu
from jax.experimental.pallas import tpu_sc as plsc

_N_FREQ = 12
_NUM_LODS = 8
_FPL = 2
_BATCH = 524288
_GRID_BASES = (16, 8)
_NLV = 8

_NC = 2
_NS = 16
_L = 16
_NW = _NC * _NS
_PTS_W = _BATCH // _NW
_CH = 128
_BLK = 1024
_NCH = _BLK // _CH
_NBLK = _PTS_W // _BLK


def _sc_feats_body(u_hbm, v_hbm, l_hbm, t0_hbm, t1_hbm,
                   o0_hbm, o1_hbm, o2_hbm, o3_hbm,
                   u_v, v_v, l_v, idx_v, val_v, w_v, out_v,
                   sem_g, sem_x, sem_o):
    wid = lax.axis_index("s") * _NC + lax.axis_index("c")
    wbase = wid * _PTS_W
    xin = ((u_hbm, u_v), (v_hbm, v_v), (l_hbm, l_v))
    outs = (o0_hbm, o1_hbm, o2_hbm, o3_hbm)

    def start_x(b, p):
        base = wbase + b * _BLK
        for hb, a in xin:
            pltpu.async_copy(hb.at[pl.ds(base, _BLK)], a.at[p], sem_x)

    def wait_x():
        for hb, a in xin:
            pltpu.make_async_copy(hb.at[pl.ds(0, _BLK)], a.at[0],
                                  sem_x).wait()

    start_x(0, 0)

    def blk(b, carry):
        p = lax.bitwise_and(b, 1)
        base = wbase + b * _BLK
        wait_x()
        start_x(jnp.minimum(b + 1, _NBLK - 1), 1 - p)

        def ph1(c, _):
            for g in range(_CH // _L):
                sl = pl.ds(c * _CH + g * _L, _L)
                slc = pl.ds(g * _L, _L)
                u = u_v[p, sl]
                v = v_v[p, sl]
                lod = l_v[p, sl]
                mips = lod * float(_NUM_LODS - 1)
                t = (float(_NLV - 1)
                     - jnp.minimum(mips, float(_NLV - 1))) * 2.0
                for gi in range(2):
                    r0 = _GRID_BASES[gi]
                    for k in range(2):
                        col = (t + float(k)).astype(jnp.int32)
                        lvl = lax.shift_right_logical(col, 1)
                        feat = lax.bitwise_and(col, 1)
                        res = lax.shift_left(
                            jnp.full((_L,), r0, jnp.int32), lvl)
                        scale = res.astype(jnp.float32) - 1.0
                        pow4m1 = lax.shift_left(
                            jnp.full((_L,), 1, jnp.int32), lvl * 2) - 1
                        offs = (r0 * r0) * lax.bitwise_and(pow4m1,
                                                           0x55555555)
                        px = u * scale + 0.5
                        py = v * scale + 0.5
                        p0xi = px.astype(jnp.int32)
                        p0yi = py.astype(jnp.int32)
                        wx = px - p0xi.astype(jnp.float32)
                        wy = py - p0yi.astype(jnp.float32)
                        rm1 = res - 1
                        p0x = jnp.clip(p0xi, 0, rm1)
                        p0y = jnp.clip(p0yi, 0, rm1)
                        p1x = jnp.minimum(p0x + 1, rm1)
                        p1y = jnp.minimum(p0y + 1, rm1)
                        rowa = offs + p0y * res
                        rowb = offs + p1y * res
                        rows = (rowa + p0x, rowa + p1x,
                                rowb + p0x, rowb + p1x)
                        r = gi * 8 + k * 4
                        for j, row in enumerate(rows):
                            e = (lax.shift_left(
                                    lax.shift_right_logical(row, 7), 8)
                                 + lax.shift_left(feat, 7)
                                 + lax.bitwise_and(row, 127))
                            idx_v[c, r + j, slc] = e
                        w = gi * 4 + k * 2
                        w_v[c, w + 0, slc] = wx
                        w_v[c, w + 1, slc] = wy
            for r in range(8):
                pltpu.async_copy(t0_hbm.at[idx_v.at[c].at[r]],
                                 val_v.at[c].at[r], sem_g)
            for r in range(8, 16):
                pltpu.async_copy(t1_hbm.at[idx_v.at[c].at[r]],
                                 val_v.at[c].at[r], sem_g)
            return _

        lax.fori_loop(0, _NCH, ph1, 0)

        def drain(c, _):
            for r in range(16):
                pltpu.make_async_copy(t0_hbm.at[idx_v.at[0].at[0]],
                                      val_v.at[0].at[0], sem_g).wait()
            return _

        lax.fori_loop(0, _NCH, drain, 0)

        @pl.when(b >= 2)
        def _wait_out():
            for j in range(4):
                pltpu.make_async_copy(out_v.at[0].at[0],
                                      o0_hbm.at[pl.ds(0, _BLK)],
                                      sem_o).wait()

        def ph2(c, _):
            for g in range(_CH // _L):
                sl = pl.ds(c * _CH + g * _L, _L)
                slc = pl.ds(g * _L, _L)
                for gi in range(2):
                    for k in range(2):
                        r = gi * 8 + k * 4
                        f00 = val_v[c, r + 0, slc]
                        f10 = val_v[c, r + 1, slc]
                        f01 = val_v[c, r + 2, slc]
                        f11 = val_v[c, r + 3, slc]
                        w = gi * 4 + k * 2
                        wx = w_v[c, w + 0, slc]
                        wy = w_v[c, w + 1, slc]
                        omx = 1.0 - wx
                        omy = 1.0 - wy
                        out_v[p, gi * 2 + k, sl] = (
                            f00 * omx * omy + f10 * wx * omy
                            + f01 * omx * wy + f11 * wx * wy)
            return _

        lax.fori_loop(0, _NCH, ph2, 0)

        for j, o_hbm in enumerate(outs):
            pltpu.async_copy(out_v.at[p].at[j],
                             o_hbm.at[pl.ds(base, _BLK)], sem_o)
        return carry

    lax.fori_loop(0, _NBLK, blk, 0)

    for _ in range(8):
        pltpu.make_async_copy(out_v.at[0].at[0], o0_hbm.at[pl.ds(0, _BLK)],
                              sem_o).wait()
    wait_x()


def _sc_feats(u, v, lod, t0, t1):
    out = jax.ShapeDtypeStruct((_BATCH,), jnp.float32)
    mesh = plsc.VectorSubcoreMesh(core_axis_name="c", subcore_axis_name="s")
    return pl.kernel(
        _sc_feats_body,
        out_type=[out, out, out, out],
        mesh=mesh,
        scratch_types=[
            pltpu.VMEM((2, _BLK), jnp.float32),
            pltpu.VMEM((2, _BLK), jnp.float32),
            pltpu.VMEM((2, _BLK), jnp.float32),
            pltpu.VMEM((_NCH, 16, _CH), jnp.int32),
            pltpu.VMEM((_NCH, 16, _CH), jnp.float32),
            pltpu.VMEM((_NCH, 8, _CH), jnp.float32),
            pltpu.VMEM((2, 4, _BLK), jnp.float32),
            pltpu.SemaphoreType.DMA,
            pltpu.SemaphoreType.DMA,
            pltpu.SemaphoreType.DMA,
        ],
    )(u, v, lod, t0, t1)


_BM = 4096


def _mlp_body(x_ref, f_ref, w0_ref, w1_ref, w2_ref, o_ref):
    x = x_ref[...]
    u = x[:, 0:1]
    v = x[:, 1:2]
    lod = x[:, 2:3]
    kf = lax.broadcasted_iota(jnp.int32, (_BM, _N_FREQ), 1).astype(jnp.float32)
    freqs = jnp.exp2(kf)

    def tri(w):
        return jnp.abs(w - jnp.floor(w + 0.5)) * 4.0 - 1.0

    h = jnp.concatenate(
        [tri(u * freqs), tri(v * freqs), f_ref[...], lod], axis=1)

    def lrelu(a):
        return jnp.where(a >= 0, a, 0.01 * a)

    h = lrelu(jnp.dot(h, w0_ref[...], preferred_element_type=jnp.float32))
    h = lrelu(jnp.dot(h, w1_ref[...], preferred_element_type=jnp.float32))
    h = lrelu(jnp.dot(h, w2_ref[...], preferred_element_type=jnp.float32))
    o_ref[...] = h


def _mlp(x, feats, w0p, w1, w2p):
    grid = (_BATCH // _BM,)
    return pl.pallas_call(
        _mlp_body,
        grid=grid,
        in_specs=[
            pl.BlockSpec((_BM, 3), lambda i: (i, 0)),
            pl.BlockSpec((_BM, 4), lambda i: (i, 0)),
            pl.BlockSpec((29, 64), lambda i: (0, 0)),
            pl.BlockSpec((64, 64), lambda i: (0, 0)),
            pl.BlockSpec((64, 8), lambda i: (0, 0)),
        ],
        out_specs=pl.BlockSpec((_BM, 8), lambda i: (i, 0)),
        out_shape=jax.ShapeDtypeStruct((_BATCH, 8), jnp.float32),
    )(x, feats, w0p, w1, w2p)


@jax.jit
def kernel(x, grid0_params, grid1_params, W0, W1, W2):
    u = x[:, 0]
    v = x[:, 1]
    lod = x[:, 2]
    def tiled_flat(t):
        n = t.shape[0]
        if n % 128:
            t = jnp.pad(t, ((0, 128 - n % 128), (0, 0)))
            n = t.shape[0]
        return t.reshape(n // 128, 128, 2).transpose(0, 2, 1).reshape(-1)

    t0 = tiled_flat(grid0_params)
    t1 = tiled_flat(grid1_params)
    f0a, f0b, f1a, f1b = _sc_feats(u, v, lod, t0, t1)
    feats = jnp.stack([f0a, f0b, f1a, f1b], axis=1)
    w2p = jnp.pad(W2, ((0, 0), (0, 5)))
    out = _mlp(x, feats, W0, W1, w2p)
    return out[:, :3]
